# baseline (speedup 1.0000x reference)
import functools

import jax
import jax.numpy as jnp
from jax.experimental import pallas as pl
from jax.experimental.pallas import tpu as pltpu


def _cnn_block_kernel(x_ref, w_ref, b_ref, o_ref, xp_ref, col_ref,
                      *, H, W, Cin, Cout):
    Ho, Wo = H // 2, W // 2
    M = H * W
    X0 = 128

    @pl.when(pl.program_id(0) == 0)
    def _():
        xp_ref[:, 0:X0] = jnp.zeros((Cin, X0), jnp.float32)
        xp_ref[:, X0 + M:X0 + M + X0] = jnp.zeros((Cin, X0), jnp.float32)

    xp_ref[:, X0:X0 + M] = x_ref[...].reshape(Cin, M)
    xp = xp_ref[...]

    lane = jax.lax.broadcasted_iota(jnp.int32, (1, M), 1) % W
    not_first = lane != 0
    not_last = lane != (W - 1)

    for dy in range(3):
        for dx in range(3):
            t = dy * 3 + dx
            s = X0 + (dy - 1) * W + (dx - 1)
            v = xp[:, s:s + M]
            if dx == 0:
                v = jnp.where(not_first, v, 0.0)
            elif dx == 2:
                v = jnp.where(not_last, v, 0.0)
            col_ref[t * Cin:(t + 1) * Cin, :] = v.astype(jnp.bfloat16)

    acc = jnp.dot(w_ref[...], col_ref[...], preferred_element_type=jnp.float32)

    at = jnp.transpose(acc, (1, 0))
    w3 = at.reshape(H * Wo, 2, Cout)
    wp = jnp.maximum(w3[:, 0, :], w3[:, 1, :])
    h4 = wp.reshape(Ho, 2, Wo, Cout)
    pooled = jnp.maximum(h4[:, 0], h4[:, 1])

    o_ref[...] = jnp.maximum(pooled + b_ref[...], 0.0)


def kernel(x_nchw, w_oihw, bias):
    B, Cin, H, W = x_nchw.shape
    Cout = w_oihw.shape[0]
    Ho, Wo = H // 2, W // 2
    K = 9 * Cin
    X0 = 128

    w_mat = jnp.transpose(w_oihw, (0, 2, 3, 1)).reshape(Cout, K)
    w_mat = w_mat.astype(jnp.bfloat16)
    b_row = bias.reshape(1, Cout).astype(jnp.float32)

    body = functools.partial(_cnn_block_kernel, H=H, W=W, Cin=Cin, Cout=Cout)
    out_nhwc = pl.pallas_call(
        body,
        out_shape=jax.ShapeDtypeStruct((B, Ho, Wo, Cout), x_nchw.dtype),
        grid=(B,),
        in_specs=[
            pl.BlockSpec((None, Cin, H, W), lambda b: (b, 0, 0, 0)),
            pl.BlockSpec((Cout, K), lambda b: (0, 0)),
            pl.BlockSpec((1, Cout), lambda b: (0, 0)),
        ],
        out_specs=pl.BlockSpec((None, Ho, Wo, Cout), lambda b: (b, 0, 0, 0)),
        scratch_shapes=[
            pltpu.VMEM((Cin, X0 + H * W + X0), jnp.float32),
            pltpu.VMEM((K, H * W), jnp.bfloat16),
        ],
        compiler_params=pltpu.CompilerParams(
            dimension_semantics=("arbitrary",),
        ),
    )(x_nchw, w_mat, b_row)

    return jnp.transpose(out_nhwc, (0, 3, 1, 2))


# speedup vs baseline: 1.1437x; 1.1437x over previous
import functools

import jax
import jax.numpy as jnp
from jax.experimental import pallas as pl
from jax.experimental.pallas import tpu as pltpu


def _cnn_block_kernel(x_ref, w_ref, b_ref, o_ref, xp_ref, col_ref,
                      *, H, W, Cin, Cout):
    Ho, Wo = H // 2, W // 2
    M = H * W
    X0 = 128

    @pl.when(pl.program_id(0) == 0)
    def _():
        xp_ref[0:X0, :] = jnp.zeros((X0, Cin), jnp.float32)
        xp_ref[X0 + M:X0 + M + X0, :] = jnp.zeros((X0, Cin), jnp.float32)

    xp_ref[X0:X0 + M, :] = x_ref[...].reshape(M, Cin)
    xp = xp_ref[...]

    row = jax.lax.broadcasted_iota(jnp.int32, (M, 1), 0) % W
    not_first = row != 0
    not_last = row != (W - 1)

    for dy in range(3):
        for dx in range(3):
            t = dy * 3 + dx
            s = X0 + (dy - 1) * W + (dx - 1)
            v = xp[s:s + M, :]
            if dx == 0:
                v = jnp.where(not_first, v, 0.0)
            elif dx == 2:
                v = jnp.where(not_last, v, 0.0)
            col_ref[:, t * Cin:(t + 1) * Cin] = v.astype(jnp.bfloat16)

    acc = jnp.dot(col_ref[...], w_ref[...], preferred_element_type=jnp.float32)

    w3 = acc.reshape(H * Wo, 2, Cout)
    wp = jnp.maximum(w3[:, 0, :], w3[:, 1, :])
    h4 = wp.reshape(Ho, 2, Wo, Cout)
    pooled = jnp.maximum(h4[:, 0], h4[:, 1])

    o_ref[...] = jnp.maximum(pooled + b_ref[...], 0.0)


def kernel(x_nchw, w_oihw, bias):
    B, Cin, H, W = x_nchw.shape
    Cout = w_oihw.shape[0]
    Ho, Wo = H // 2, W // 2
    K = 9 * Cin
    X0 = 128

    x_nhwc = jnp.transpose(x_nchw, (0, 2, 3, 1))
    w_mat = jnp.transpose(w_oihw, (2, 3, 1, 0)).reshape(K, Cout)
    w_mat = w_mat.astype(jnp.bfloat16)
    b_row = bias.reshape(1, Cout).astype(jnp.float32)

    body = functools.partial(_cnn_block_kernel, H=H, W=W, Cin=Cin, Cout=Cout)
    out_nhwc = pl.pallas_call(
        body,
        out_shape=jax.ShapeDtypeStruct((B, Ho, Wo, Cout), x_nchw.dtype),
        grid=(B,),
        in_specs=[
            pl.BlockSpec((None, H, W, Cin), lambda b: (b, 0, 0, 0)),
            pl.BlockSpec((K, Cout), lambda b: (0, 0)),
            pl.BlockSpec((1, Cout), lambda b: (0, 0)),
        ],
        out_specs=pl.BlockSpec((None, Ho, Wo, Cout), lambda b: (b, 0, 0, 0)),
        scratch_shapes=[
            pltpu.VMEM((X0 + H * W + X0, Cin), jnp.float32),
            pltpu.VMEM((H * W, K), jnp.bfloat16),
        ],
        compiler_params=pltpu.CompilerParams(
            dimension_semantics=("arbitrary",),
        ),
    )(x_nhwc, w_mat, b_row)

    return jnp.transpose(out_nhwc, (0, 3, 1, 2))


# speedup vs baseline: 1.2334x; 1.0785x over previous
import functools

import jax
import jax.numpy as jnp
from jax.experimental import pallas as pl
from jax.experimental.pallas import tpu as pltpu

_IMGS = 2


def _cnn_block_kernel(x_ref, w_ref, b_ref, o_ref, xp_ref, col_ref,
                      *, H, W, Cin, Cout):
    Ho, Wo = H // 2, W // 2
    M = H * W
    X0 = 128
    G = 128

    @pl.when(pl.program_id(0) == 0)
    def _():
        xp_ref[0:X0, :] = jnp.zeros((X0, Cin), jnp.float32)
        xp_ref[X0 + M:X0 + M + G, :] = jnp.zeros((G, Cin), jnp.float32)
        xp_ref[X0 + 2 * M + G:X0 + 2 * M + G + X0, :] = (
            jnp.zeros((X0, Cin), jnp.float32))

    base = (X0, X0 + M + G)
    xp_ref[base[0]:base[0] + M, :] = x_ref[0].reshape(M, Cin)
    xp_ref[base[1]:base[1] + M, :] = x_ref[1].reshape(M, Cin)
    xp = xp_ref[...]

    row = jax.lax.broadcasted_iota(jnp.int32, (M, 1), 0) % W
    not_first = row != 0
    not_last = row != (W - 1)

    for i in range(2):
        for dy in range(3):
            for dx in range(3):
                t = dy * 3 + dx
                s = base[i] + (dy - 1) * W + (dx - 1)
                v = xp[s:s + M, :]
                if dx == 0:
                    v = jnp.where(not_first, v, 0.0)
                elif dx == 2:
                    v = jnp.where(not_last, v, 0.0)
                col_ref[i * M:(i + 1) * M, t * Cin:(t + 1) * Cin] = (
                    v.astype(jnp.bfloat16))

    acc = jnp.dot(col_ref[...], w_ref[...], preferred_element_type=jnp.float32)

    w3 = acc.reshape(H * Wo * 2, 2, Cout)
    wp = jnp.maximum(w3[:, 0, :], w3[:, 1, :])
    h4 = wp.reshape(2 * Ho, 2, Wo, Cout)
    pooled = jnp.maximum(h4[:, 0], h4[:, 1])

    out = jnp.maximum(pooled + b_ref[...], 0.0)
    o_ref[...] = out.reshape(2, Ho, Wo, Cout)


def kernel(x_nchw, w_oihw, bias):
    B, Cin, H, W = x_nchw.shape
    Cout = w_oihw.shape[0]
    Ho, Wo = H // 2, W // 2
    K = 9 * Cin
    X0 = 128
    G = 128
    M = H * W

    x_nhwc = jnp.transpose(x_nchw, (0, 2, 3, 1))
    w_mat = jnp.transpose(w_oihw, (2, 3, 1, 0)).reshape(K, Cout)
    w_mat = w_mat.astype(jnp.bfloat16)
    b_row = bias.reshape(1, Cout).astype(jnp.float32)

    body = functools.partial(_cnn_block_kernel, H=H, W=W, Cin=Cin, Cout=Cout)
    out_nhwc = pl.pallas_call(
        body,
        out_shape=jax.ShapeDtypeStruct((B, Ho, Wo, Cout), x_nchw.dtype),
        grid=(B // _IMGS,),
        in_specs=[
            pl.BlockSpec((_IMGS, H, W, Cin), lambda b: (b, 0, 0, 0)),
            pl.BlockSpec((K, Cout), lambda b: (0, 0)),
            pl.BlockSpec((1, Cout), lambda b: (0, 0)),
        ],
        out_specs=pl.BlockSpec((_IMGS, Ho, Wo, Cout), lambda b: (b, 0, 0, 0)),
        scratch_shapes=[
            pltpu.VMEM((X0 + 2 * M + G + X0, Cin), jnp.float32),
            pltpu.VMEM((2 * M, K), jnp.bfloat16),
        ],
        compiler_params=pltpu.CompilerParams(
            dimension_semantics=("arbitrary",),
        ),
    )(x_nhwc, w_mat, b_row)

    return jnp.transpose(out_nhwc, (0, 3, 1, 2))


# speedup vs baseline: 1.2961x; 1.0508x over previous
import functools

import jax
import jax.numpy as jnp
from jax.experimental import pallas as pl
from jax.experimental.pallas import tpu as pltpu

_IMGS = 4


def _cnn_block_kernel(x_ref, w_ref, b_ref, o_ref, xp_ref, col_ref,
                      *, H, W, Cin, Cout):
    Ho, Wo = H // 2, W // 2
    M = H * W
    X0 = 128
    G = 128

    base = tuple(X0 + i * (M + G) for i in range(_IMGS))

    @pl.when(pl.program_id(0) == 0)
    def _():
        xp_ref[0:X0, :] = jnp.zeros((X0, Cin), jnp.float32)
        for i in range(_IMGS):
            xp_ref[base[i] + M:base[i] + M + G, :] = (
                jnp.zeros((G, Cin), jnp.float32))

    for i in range(_IMGS):
        xp_ref[base[i]:base[i] + M, :] = x_ref[i].reshape(M, Cin)
    xp = xp_ref[...]

    row = jax.lax.broadcasted_iota(jnp.int32, (M, 1), 0) % W
    not_first = row != 0
    not_last = row != (W - 1)

    for i in range(_IMGS):
        for dy in range(3):
            for dx in range(3):
                t = dy * 3 + dx
                s = base[i] + (dy - 1) * W + (dx - 1)
                v = xp[s:s + M, :]
                if dx == 0:
                    v = jnp.where(not_first, v, 0.0)
                elif dx == 2:
                    v = jnp.where(not_last, v, 0.0)
                col_ref[i * M:(i + 1) * M, t * Cin:(t + 1) * Cin] = (
                    v.astype(jnp.bfloat16))

    acc = jnp.dot(col_ref[...], w_ref[...], preferred_element_type=jnp.float32)

    w3 = acc.reshape(H * Wo * _IMGS, 2, Cout)
    wp = jnp.maximum(w3[:, 0, :], w3[:, 1, :])
    h4 = wp.reshape(_IMGS * Ho, 2, Wo, Cout)
    pooled = jnp.maximum(h4[:, 0], h4[:, 1])

    out = jnp.maximum(pooled + b_ref[...], 0.0)
    o_ref[...] = out.reshape(_IMGS, Ho, Wo, Cout)


def kernel(x_nchw, w_oihw, bias):
    B, Cin, H, W = x_nchw.shape
    Cout = w_oihw.shape[0]
    Ho, Wo = H // 2, W // 2
    K = 9 * Cin
    X0 = 128
    G = 128
    M = H * W

    x_nhwc = jnp.transpose(x_nchw, (0, 2, 3, 1))
    w_mat = jnp.transpose(w_oihw, (2, 3, 1, 0)).reshape(K, Cout)
    w_mat = w_mat.astype(jnp.bfloat16)
    b_row = bias.reshape(1, Cout).astype(jnp.float32)

    body = functools.partial(_cnn_block_kernel, H=H, W=W, Cin=Cin, Cout=Cout)
    out_nhwc = pl.pallas_call(
        body,
        out_shape=jax.ShapeDtypeStruct((B, Ho, Wo, Cout), x_nchw.dtype),
        grid=(B // _IMGS,),
        in_specs=[
            pl.BlockSpec((_IMGS, H, W, Cin), lambda b: (b, 0, 0, 0)),
            pl.BlockSpec((K, Cout), lambda b: (0, 0)),
            pl.BlockSpec((1, Cout), lambda b: (0, 0)),
        ],
        out_specs=pl.BlockSpec((_IMGS, Ho, Wo, Cout), lambda b: (b, 0, 0, 0)),
        scratch_shapes=[
            pltpu.VMEM((X0 + _IMGS * (M + G), Cin), jnp.float32),
            pltpu.VMEM((_IMGS * M, K), jnp.bfloat16),
        ],
        compiler_params=pltpu.CompilerParams(
            dimension_semantics=("arbitrary",),
        ),
    )(x_nhwc, w_mat, b_row)

    return jnp.transpose(out_nhwc, (0, 3, 1, 2))
